# initial kernel scaffold (unmeasured)
import jax
import jax.numpy as jnp
from jax import lax
from jax.experimental import pallas as pl
from jax.experimental.pallas import tpu as pltpu

N_DEV = 4
EPS = 1e-5
LANES = 128


def kernel(x, gamma):
    m, n_local = x.shape
    n_global = n_local * N_DEV
    sub = m // LANES

    gamma2 = gamma.reshape(1, n_local)

    def body(x_ref, g_ref, out_ref, comm_ref, send_sems, recv_sems):
        my = lax.axis_index("i")

        xf = x_ref[:, :].astype(jnp.float32)
        p = jnp.sum(xf * xf, axis=1)
        comm_ref[pl.ds(my, 1)] = p.reshape(1, sub, LANES)

        sends = []
        for d in range(1, N_DEV):
            dst = lax.rem(my + d, N_DEV)
            rdma = pltpu.make_async_remote_copy(
                src_ref=comm_ref.at[my],
                dst_ref=comm_ref.at[my],
                send_sem=send_sems.at[d - 1],
                recv_sem=recv_sems.at[my],
                device_id=(dst,),
                device_id_type=pl.DeviceIdType.MESH,
            )
            rdma.start()
            sends.append(rdma)

        for d in range(1, N_DEV):
            src = lax.rem(my + d, N_DEV)
            recv = pltpu.make_async_remote_copy(
                src_ref=comm_ref.at[src],
                dst_ref=comm_ref.at[src],
                send_sem=send_sems.at[d - 1],
                recv_sem=recv_sems.at[src],
                device_id=(my,),
                device_id_type=pl.DeviceIdType.MESH,
            )
            recv.wait_recv()

        total = comm_ref[0] + comm_ref[1] + comm_ref[2] + comm_ref[3]
        inv = lax.rsqrt(total / n_global + EPS)
        inv_col = inv.reshape(m, 1)
        out_ref[:, :] = xf * inv_col * g_ref[:, :].astype(jnp.float32)

        for rdma in sends:
            rdma.wait_send()

    return pl.pallas_call(
        body,
        out_shape=jax.ShapeDtypeStruct((m, n_local), jnp.float32),
        in_specs=[
            pl.BlockSpec(memory_space=pltpu.VMEM),
            pl.BlockSpec(memory_space=pltpu.VMEM),
        ],
        out_specs=pl.BlockSpec(memory_space=pltpu.VMEM),
        scratch_shapes=[
            pltpu.VMEM((N_DEV, sub, LANES), jnp.float32),
            pltpu.SemaphoreType.DMA((N_DEV - 1,)),
            pltpu.SemaphoreType.DMA((N_DEV,)),
        ],
        compiler_params=pltpu.CompilerParams(collective_id=0),
    )(x, gamma2)


# baseline (device time: 36028 ns/iter reference)
import jax
import jax.numpy as jnp
from jax import lax
from jax.experimental import pallas as pl
from jax.experimental.pallas import tpu as pltpu

N_DEV = 4
EPS = 1e-5
LANES = 128


def kernel(x, gamma):
    m, n_local = x.shape
    n_global = n_local * N_DEV
    sub = m // LANES

    gamma2 = gamma.reshape(1, n_local)

    def body(x_ref, g_ref, out_ref, comm_ref, send_sems, recv_sems):
        my = lax.axis_index("i")

        xf = x_ref[:, :].astype(jnp.float32)
        p_col = jnp.sum(xf * xf, axis=1, keepdims=True)

        ri = lax.broadcasted_iota(jnp.int32, (m, LANES), 0)
        li = lax.broadcasted_iota(jnp.int32, (m, LANES), 1)
        mask = (jnp.bitwise_and(ri, LANES - 1) == li).astype(jnp.float32)
        si_r = lax.broadcasted_iota(jnp.int32, (sub, m), 0)
        ri_r = lax.broadcasted_iota(jnp.int32, (sub, m), 1)
        lt = (jnp.right_shift(ri_r, 7) == si_r).astype(jnp.float32)
        ri_c = lax.broadcasted_iota(jnp.int32, (m, sub), 0)
        si_c = lax.broadcasted_iota(jnp.int32, (m, sub), 1)
        lsel = (jnp.right_shift(ri_c, 7) == si_c).astype(jnp.float32)

        packed = jnp.dot(lt, p_col * mask, preferred_element_type=jnp.float32)
        comm_ref[pl.ds(my, 1)] = packed[None]

        sends = []
        for d in range(1, N_DEV):
            dst = lax.rem(my + d, N_DEV)
            rdma = pltpu.make_async_remote_copy(
                src_ref=comm_ref.at[my],
                dst_ref=comm_ref.at[my],
                send_sem=send_sems.at[d - 1],
                recv_sem=recv_sems.at[my],
                device_id=(dst,),
                device_id_type=pl.DeviceIdType.MESH,
            )
            rdma.start()
            sends.append(rdma)

        for d in range(1, N_DEV):
            src = lax.rem(my + d, N_DEV)
            recv = pltpu.make_async_remote_copy(
                src_ref=comm_ref.at[src],
                dst_ref=comm_ref.at[src],
                send_sem=send_sems.at[d - 1],
                recv_sem=recv_sems.at[src],
                device_id=(my,),
                device_id_type=pl.DeviceIdType.MESH,
            )
            recv.wait_recv()

        total = comm_ref[0] + comm_ref[1] + comm_ref[2] + comm_ref[3]
        inv_packed = lax.rsqrt(total / n_global + EPS)
        a = jnp.dot(lsel, inv_packed, preferred_element_type=jnp.float32)
        inv_col = jnp.sum(a * mask, axis=1, keepdims=True)

        out_ref[:, :] = xf * inv_col * g_ref[:, :].astype(jnp.float32)

        for rdma in sends:
            rdma.wait_send()

    return pl.pallas_call(
        body,
        out_shape=jax.ShapeDtypeStruct((m, n_local), jnp.float32),
        in_specs=[
            pl.BlockSpec(memory_space=pltpu.VMEM),
            pl.BlockSpec(memory_space=pltpu.VMEM),
        ],
        out_specs=pl.BlockSpec(memory_space=pltpu.VMEM),
        scratch_shapes=[
            pltpu.VMEM((N_DEV, sub, LANES), jnp.float32),
            pltpu.SemaphoreType.DMA((N_DEV - 1,)),
            pltpu.SemaphoreType.DMA((N_DEV,)),
        ],
        compiler_params=pltpu.CompilerParams(
            vmem_limit_bytes=64 * 1024 * 1024,
        ),
    )(x, gamma2)


# device time: 27851 ns/iter; 1.2936x vs baseline; 1.2936x over previous
import jax
import jax.numpy as jnp
from jax import lax
from jax.experimental import pallas as pl
from jax.experimental.pallas import tpu as pltpu

N_DEV = 4
EPS = 1e-5
LANES = 128


def kernel(x, gamma):
    m, n_local = x.shape
    n_global = n_local * N_DEV
    sub = m // LANES

    gamma2 = gamma.reshape(1, n_local)

    def body(x_ref, g_ref, out_ref, comm_ref, send_sems, recv_sems):
        my = lax.axis_index("i")

        xf = x_ref[:, :].astype(jnp.float32)
        p_col = jnp.sum(xf * xf, axis=1, keepdims=True)

        ri = lax.broadcasted_iota(jnp.int32, (m, LANES), 0)
        li = lax.broadcasted_iota(jnp.int32, (m, LANES), 1)
        mask = (jnp.bitwise_and(ri, LANES - 1) == li).astype(jnp.float32)
        si_r = lax.broadcasted_iota(jnp.int32, (sub, m), 0)
        ri_r = lax.broadcasted_iota(jnp.int32, (sub, m), 1)
        lt = (jnp.right_shift(ri_r, 7) == si_r).astype(jnp.float32)

        packed = jnp.dot(lt, p_col * mask, preferred_element_type=jnp.float32)
        comm_ref[pl.ds(my, 1)] = packed[None]

        sends = []
        for d in range(1, N_DEV):
            dst = lax.rem(my + d, N_DEV)
            rdma = pltpu.make_async_remote_copy(
                src_ref=comm_ref.at[my],
                dst_ref=comm_ref.at[my],
                send_sem=send_sems.at[d - 1],
                recv_sem=recv_sems.at[my],
                device_id=(dst,),
                device_id_type=pl.DeviceIdType.MESH,
            )
            rdma.start()
            sends.append(rdma)

        pre = xf * g_ref[:, :].astype(jnp.float32)
        ri_c = lax.broadcasted_iota(jnp.int32, (m, sub), 0)
        si_c = lax.broadcasted_iota(jnp.int32, (m, sub), 1)
        lsel = (jnp.right_shift(ri_c, 7) == si_c).astype(jnp.float32)

        for d in range(1, N_DEV):
            src = lax.rem(my + d, N_DEV)
            recv = pltpu.make_async_remote_copy(
                src_ref=comm_ref.at[src],
                dst_ref=comm_ref.at[src],
                send_sem=send_sems.at[d - 1],
                recv_sem=recv_sems.at[src],
                device_id=(my,),
                device_id_type=pl.DeviceIdType.MESH,
            )
            recv.wait_recv()

        total = comm_ref[0] + comm_ref[1] + comm_ref[2] + comm_ref[3]
        inv_packed = lax.rsqrt(total / n_global + EPS)
        a = jnp.dot(lsel, inv_packed, preferred_element_type=jnp.float32)
        inv_col = jnp.sum(a * mask, axis=1, keepdims=True)

        out_ref[:, :] = (pre * inv_col).astype(jnp.bfloat16)

        for rdma in sends:
            rdma.wait_send()

    return pl.pallas_call(
        body,
        out_shape=jax.ShapeDtypeStruct((m, n_local), jnp.bfloat16),
        in_specs=[
            pl.BlockSpec(memory_space=pltpu.VMEM),
            pl.BlockSpec(memory_space=pltpu.VMEM),
        ],
        out_specs=pl.BlockSpec(memory_space=pltpu.VMEM),
        scratch_shapes=[
            pltpu.VMEM((N_DEV, sub, LANES), jnp.float32),
            pltpu.SemaphoreType.DMA((N_DEV - 1,)),
            pltpu.SemaphoreType.DMA((N_DEV,)),
        ],
        compiler_params=pltpu.CompilerParams(
            vmem_limit_bytes=64 * 1024 * 1024,
        ),
    )(x, gamma2)
